# baseline (speedup 1.0000x reference)
import functools

import jax
import jax.numpy as jnp
from jax import lax
from jax.experimental import pallas as pl
from jax.experimental.pallas import tpu as pltpu
from jax.experimental.pallas import tpu_sc as plsc

BLK = 256


def _router_body(x_ref, rw_ref, eidx_ref, gwt_ref):
    logits = jax.lax.dot_general(
        rw_ref[...], x_ref[...], (((1,), (1,)), ((), ())),
        preferred_element_type=jnp.float32)
    m = jnp.max(logits, axis=0, keepdims=True)
    p = jnp.exp(logits - m)
    probs = p / jnp.sum(p, axis=0, keepdims=True)
    E = probs.shape[0]
    sub = jax.lax.broadcasted_iota(jnp.int32, probs.shape, 0)
    m1 = jnp.max(probs, axis=0, keepdims=True)
    a1 = jnp.min(jnp.where(probs == m1, sub, E), axis=0, keepdims=True)
    p2 = jnp.where(sub == a1, -1.0, probs)
    m2 = jnp.max(p2, axis=0, keepdims=True)
    a2 = jnp.min(jnp.where(p2 == m2, sub, E), axis=0, keepdims=True)
    den = m1 + m2 + 1e-9
    eidx_ref[...] = jnp.concatenate([a1, a2], axis=0)
    gwt_ref[...] = jnp.concatenate([m1 / den, m2 / den], axis=0)


def _router_tc(x_flat, router_w):
    T, D = x_flat.shape
    E = router_w.shape[0]
    BT = min(512, T)
    NT = T // BT
    return pl.pallas_call(
        _router_body,
        grid=(NT,),
        in_specs=[
            pl.BlockSpec((BT, D), lambda t: (t, 0)),
            pl.BlockSpec((E, D), lambda t: (0, 0)),
        ],
        out_specs=[
            pl.BlockSpec((2, BT), lambda t: (0, t)),
            pl.BlockSpec((2, BT), lambda t: (0, t)),
        ],
        out_shape=[
            jax.ShapeDtypeStruct((2, T), jnp.int32),
            jax.ShapeDtypeStruct((2, T), jnp.float32),
        ],
    )(x_flat, router_w)


def _sc_mesh():
    info = plsc.get_sparse_core_info()
    nw = info.num_cores * info.num_subcores
    mesh = plsc.VectorSubcoreMesh(core_axis_name="c", subcore_axis_name="s")
    return info, nw, mesh


def _wid(info):
    return lax.axis_index("s") * info.num_cores + lax.axis_index("c")


def _hist_sc(eflat, E):
    A = eflat.shape[0]
    info, NW, mesh = _sc_mesh()
    CH = A // NW

    @functools.partial(
        pl.kernel, mesh=mesh,
        compiler_params=pltpu.CompilerParams(needs_layout_passes=False),
        out_type=jax.ShapeDtypeStruct((NW, 16), jnp.int32),
        scratch_types=[
            pltpu.VMEM((CH,), jnp.int32),
            pltpu.VMEM((16,), jnp.int32),
        ],
    )
    def hist_k(eidx_hbm, hist_hbm, eid_v, hbuf):
        w = _wid(info)
        pltpu.sync_copy(eidx_hbm.at[pl.ds(w * CH, CH)], eid_v)

        def step(i, accs):
            d = eid_v[pl.ds(i * 16, 16)]
            return tuple(a + (d == jnp.int32(e)).astype(jnp.int32)
                         for e, a in enumerate(accs))

        accs = lax.fori_loop(0, CH // 16, step,
                             tuple(jnp.zeros((16,), jnp.int32)
                                   for _ in range(E)))
        iota = lax.iota(jnp.int32, 16)
        hist = jnp.zeros((16,), jnp.int32)
        for e in range(E):
            hist = jnp.where(iota == e, jnp.sum(accs[e]), hist)
        hbuf[...] = hist
        pltpu.sync_copy(hbuf, hist_hbm.at[w])

    return hist_k(eflat)


def _plan_sc(eflat, gflat, hist, x_flat, T, E, P, G16):
    A = eflat.shape[0]
    D = x_flat.shape[1]
    info, NW, mesh = _sc_mesh()
    CH = A // NW
    NV = CH // 16
    RC = 32
    NCHK = CH // RC
    assert T % CH == 0

    @functools.partial(
        pl.kernel, mesh=mesh,
        compiler_params=pltpu.CompilerParams(needs_layout_passes=False),
        out_type=(
            jax.ShapeDtypeStruct((P, D), jnp.float32),
            jax.ShapeDtypeStruct((P,), jnp.float32),
            jax.ShapeDtypeStruct((G16,), jnp.int32),
            jax.ShapeDtypeStruct((A,), jnp.int32),
        ),
        scratch_types=[
            pltpu.VMEM((CH,), jnp.int32),
            pltpu.VMEM((CH,), jnp.float32),
            pltpu.VMEM((NW, 16), jnp.int32),
            pltpu.VMEM((NCHK, RC), jnp.int32),
            pltpu.VMEM((NCHK, RC), jnp.float32),
            pltpu.VMEM((CH,), jnp.int32),
            pltpu.VMEM((G16,), jnp.int32),
            pltpu.VMEM((16,), jnp.int32),
            pltpu.VMEM((2, RC, D), jnp.float32),
            pltpu.SemaphoreType.DMA,
            pltpu.SemaphoreType.DMA,
            pltpu.SemaphoreType.DMA,
        ],
    )
    def plan_k(eidx_hbm, gwt_hbm, hist_hbm, x_hbm, xs_hbm, gw_hbm, blk_hbm,
               pos_hbm, eid_v, gwv, hall, posb, gwb, pos_lin, blkb, runbuf,
               xrow, sem_s, sem_in, sem_out):
        w = _wid(info)
        base = w * CH
        pltpu.sync_copy(eidx_hbm.at[pl.ds(base, CH)], eid_v)
        pltpu.sync_copy(gwt_hbm.at[pl.ds(base, CH)], gwv)
        pltpu.sync_copy(hist_hbm, hall)
        iota = lax.iota(jnp.int32, 16)
        total = jnp.zeros((16,), jnp.int32)
        prior = jnp.zeros((16,), jnp.int32)
        for i in range(NW):
            h = hall[i, :]
            total = total + h
            prior = prior + jnp.where(jnp.int32(i) < w, h, 0)
        padded = ((total + (BLK - 1)) // BLK) * BLK
        bases = plsc.cumsum(padded) - padded
        run = bases + prior

        @pl.when(w == 0)
        def _blocks():
            blkstart = bases // BLK
            blkcnt = padded // BLK
            for j in range(G16 // 16):
                bid = iota + j * 16
                acc = jnp.full((16,), -1, jnp.int32)
                for e in range(E):
                    s_e = jnp.sum(jnp.where(iota == e, blkstart, 0))
                    n_e = jnp.sum(jnp.where(iota == e, blkcnt, 0))
                    mb = (bid >= s_e) & (bid < s_e + n_e)
                    acc = jnp.where(mb, e, acc)
                blkb[pl.ds(j * 16, 16)] = acc
            pltpu.sync_copy(blkb, blk_hbm)

        for i in range(NV):
            d = eid_v[pl.ds(i * 16, 16)]
            runbuf[...] = run
            off = plsc.load_gather(runbuf, [d])
            pos_v = off
            for e in range(E):
                m = d == jnp.int32(e)
                mi = m.astype(jnp.int32)
                ranks = plsc.cumsum(mi) - mi
                pos_v = pos_v + jnp.where(m, ranks, 0)
                pc = plsc.all_reduce_population_count(m)
                run = run + jnp.where(iota == e, pc, 0)
            r, c = divmod(i * 16, RC)
            posb[r, pl.ds(c, 16)] = pos_v
            gwb[r, pl.ds(c, 16)] = gwv[pl.ds(i * 16, 16)]
            pos_lin[pl.ds(i * 16, 16)] = pos_v
        pltpu.sync_copy(pos_lin, pos_hbm.at[pl.ds(base, CH)])
        cps = [pltpu.async_copy(gwb.at[j], gw_hbm.at[posb.at[j]], sem_s)
               for j in range(NCHK)]
        tok0 = jnp.where(base < T, base, base - T)
        starts = []
        for c in range(min(2, NCHK)):
            starts.append(pltpu.async_copy(
                x_hbm.at[pl.ds(tok0 + c * RC, RC)], xrow.at[c % 2], sem_in))
        for c in range(NCHK):
            starts[c].wait()
            pltpu.async_copy(xrow.at[c % 2], xs_hbm.at[posb.at[c]],
                             sem_out).wait()
            if c + 2 < NCHK:
                starts.append(pltpu.async_copy(
                    x_hbm.at[pl.ds(tok0 + (c + 2) * RC, RC)],
                    xrow.at[c % 2], sem_in))
        for cp in cps:
            cp.wait()

    return plan_k(eflat, gflat, hist, x_flat)


def _gather_sc(src, idx):
    R, D = src.shape
    M = idx.shape[0]
    info, NW, mesh = _sc_mesh()
    RW = M // NW
    CH = min(32, RW)
    STEPS = RW // CH

    @functools.partial(
        pl.kernel, mesh=mesh,
        compiler_params=pltpu.CompilerParams(needs_layout_passes=False),
        out_type=jax.ShapeDtypeStruct((M, D), jnp.float32),
        scratch_types=[
            pltpu.VMEM((STEPS, CH), jnp.int32),
            pltpu.VMEM((2, CH, D), jnp.float32),
            pltpu.SemaphoreType.DMA,
            pltpu.SemaphoreType.DMA,
        ],
    )
    def gather_k(src_hbm, idx_hbm, out_hbm, idx_v, rows_v, sem_g, sem_o):
        w = _wid(info)
        for c in range(STEPS):
            pltpu.sync_copy(idx_hbm.at[pl.ds(w * RW + c * CH, CH)],
                            idx_v.at[c])
        gath = []
        for c in range(min(2, STEPS)):
            gath.append(pltpu.async_copy(src_hbm.at[idx_v.at[c]],
                                         rows_v.at[c % 2], sem_g))
        for c in range(STEPS):
            gath[c].wait()
            wb = pltpu.async_copy(rows_v.at[c % 2],
                                  out_hbm.at[pl.ds(w * RW + c * CH, CH)],
                                  sem_o)
            wb.wait()
            if c + 2 < STEPS:
                gath.append(pltpu.async_copy(src_hbm.at[idx_v.at[c + 2]],
                                             rows_v.at[c % 2], sem_g))

    return gather_k(src, idx)


def _mlp_body(bexp_ref, xs_ref, w1_ref, b1_ref, w2_ref, b2_ref, gw_ref,
              os_ref):
    b = pl.program_id(0)
    e = bexp_ref[b]

    @pl.when(e >= 0)
    def _():
        gw = gw_ref[...]
        h = jax.lax.dot_general(
            xs_ref[...], w1_ref[0],
            (((1,), (1,)), ((), ())), preferred_element_type=jnp.float32)
        h = jnp.maximum(h + b1_ref[0], 0.0) * gw
        os_ref[...] = jax.lax.dot_general(
            h, w2_ref[0], (((1,), (1,)), ((), ())),
            preferred_element_type=jnp.float32) + gw * b2_ref[0]


def _mlp_tc(xs, blk_exp, gw_sorted, w1, b1, w2, b2):
    P, D = xs.shape
    E, F, _ = w1.shape
    G = P // BLK
    grid_spec = pltpu.PrefetchScalarGridSpec(
        num_scalar_prefetch=1,
        grid=(G,),
        in_specs=[
            pl.BlockSpec((BLK, D), lambda b, s: (b, 0)),
            pl.BlockSpec((1, F, D), lambda b, s: (jnp.maximum(s[b], 0), 0, 0)),
            pl.BlockSpec((1, 1, F), lambda b, s: (jnp.maximum(s[b], 0), 0, 0)),
            pl.BlockSpec((1, D, F), lambda b, s: (jnp.maximum(s[b], 0), 0, 0)),
            pl.BlockSpec((1, 1, D), lambda b, s: (jnp.maximum(s[b], 0), 0, 0)),
            pl.BlockSpec((BLK, 1), lambda b, s: (b, 0)),
        ],
        out_specs=pl.BlockSpec((BLK, D), lambda b, s: (b, 0)),
    )
    return pl.pallas_call(
        _mlp_body,
        grid_spec=grid_spec,
        out_shape=jax.ShapeDtypeStruct((P, D), jnp.float32),
    )(blk_exp, xs, w1, b1.reshape(E, 1, F),
      w2, b2.reshape(E, 1, D),
      gw_sorted.reshape(P, 1))


def _combine_sc(os_, pos, T):
    P, D = os_.shape
    info, NW, mesh = _sc_mesh()
    TW = T // NW
    RC = 16
    STEPS = TW // RC

    @functools.partial(
        pl.kernel, mesh=mesh,
        compiler_params=pltpu.CompilerParams(needs_layout_passes=False),
        out_type=jax.ShapeDtypeStruct((T, D), jnp.float32),
        scratch_types=[
            pltpu.VMEM((STEPS, RC), jnp.int32),
            pltpu.VMEM((STEPS, RC), jnp.int32),
            pltpu.VMEM((2, RC, D), jnp.float32),
            pltpu.VMEM((2, RC, D), jnp.float32),
            pltpu.SemaphoreType.DMA,
            pltpu.SemaphoreType.DMA,
            pltpu.SemaphoreType.DMA,
        ],
    )
    def comb_k(os_hbm, pos_hbm, out_hbm, idxa, idxb, bufa, bufb,
               sem_a, sem_b, sem_o):
        w = _wid(info)
        tok0 = w * TW
        for c in range(STEPS):
            pltpu.sync_copy(pos_hbm.at[pl.ds(tok0 + c * RC, RC)],
                            idxa.at[c])
            pltpu.sync_copy(pos_hbm.at[pl.ds(T + tok0 + c * RC, RC)],
                            idxb.at[c])
        ga, gb = [], []
        for c in range(min(2, STEPS)):
            ga.append(pltpu.async_copy(os_hbm.at[idxa.at[c]],
                                       bufa.at[c % 2], sem_a))
            gb.append(pltpu.async_copy(os_hbm.at[idxb.at[c]],
                                       bufb.at[c % 2], sem_b))
        for c in range(STEPS):
            cc = c % 2
            ga[c].wait()
            gb[c].wait()
            for r in range(RC):
                def vstep(v, _, cc=cc, r=r):
                    sl = pl.ds(v * 16, 16)
                    bufa[cc, r, sl] = bufa[cc, r, sl] + bufb[cc, r, sl]
                    return 0
                lax.fori_loop(0, D // 16, vstep, 0)
            pltpu.async_copy(bufa.at[cc],
                             out_hbm.at[pl.ds(tok0 + c * RC, RC)],
                             sem_o).wait()
            if c + 2 < STEPS:
                ga.append(pltpu.async_copy(os_hbm.at[idxa.at[c + 2]],
                                           bufa.at[cc], sem_a))
                gb.append(pltpu.async_copy(os_hbm.at[idxb.at[c + 2]],
                                           bufb.at[cc], sem_b))

    return comb_k(os_, pos)


def _add_body(a_ref, b_ref, o_ref):
    o_ref[...] = a_ref[0] + b_ref[0]


def _add_tc(comb):
    _, T, D = comb.shape
    BT = min(1024, T)
    NT = T // BT
    return pl.pallas_call(
        _add_body,
        grid=(NT,),
        in_specs=[
            pl.BlockSpec((1, BT, D), lambda t: (0, t, 0)),
            pl.BlockSpec((1, BT, D), lambda t: (1, t, 0)),
        ],
        out_specs=pl.BlockSpec((BT, D), lambda t: (t, 0)),
        out_shape=jax.ShapeDtypeStruct((T, D), jnp.float32),
    )(comb, comb)


def kernel(x, router_w, w1, b1, w2, b2):
    B, N, D = x.shape
    E = router_w.shape[0]
    T = B * N
    A = 2 * T
    P = A + E * BLK
    G16 = ((P // BLK + 15) // 16) * 16
    x_flat = x.reshape(T, D)
    eidx, gwt = _router_tc(x_flat, router_w)
    eflat = eidx.reshape(A)
    gflat = gwt.reshape(A)
    hist = _hist_sc(eflat, E)
    xs, gw_s, blk_e, pos = _plan_sc(eflat, gflat, hist, x_flat, T, E, P, G16)
    os_ = _mlp_tc(xs, blk_e, gw_s, w1, b1, w2, b2)
    out = _combine_sc(os_, pos, T)
    return out.reshape(B, N, D)


# speedup vs baseline: 1.0437x; 1.0437x over previous
import functools

import jax
import jax.numpy as jnp
from jax import lax
from jax.experimental import pallas as pl
from jax.experimental.pallas import tpu as pltpu
from jax.experimental.pallas import tpu_sc as plsc

BLK = 256


def _router_body(x_ref, rw_ref, eidx_ref, gwt_ref, hist_ref):
    logits = jax.lax.dot_general(
        rw_ref[...], x_ref[...], (((1,), (1,)), ((), ())),
        preferred_element_type=jnp.float32)
    m = jnp.max(logits, axis=0, keepdims=True)
    p = jnp.exp(logits - m)
    probs = p / jnp.sum(p, axis=0, keepdims=True)
    E = probs.shape[0]
    sub = jax.lax.broadcasted_iota(jnp.int32, probs.shape, 0)
    m1 = jnp.max(probs, axis=0, keepdims=True)
    a1 = jnp.min(jnp.where(probs == m1, sub, E), axis=0, keepdims=True)
    p2 = jnp.where(sub == a1, -1.0, probs)
    m2 = jnp.max(p2, axis=0, keepdims=True)
    a2 = jnp.min(jnp.where(p2 == m2, sub, E), axis=0, keepdims=True)
    den = m1 + m2 + 1e-9
    eidx_ref[...] = jnp.concatenate([a1, a2], axis=0)
    gwt_ref[...] = jnp.concatenate([m1 / den, m2 / den], axis=0)
    BT = probs.shape[1]
    lane = jax.lax.broadcasted_iota(jnp.int32, (1, BT), 1)
    pieces = []
    for a in (a1, a2):
        oneh = (sub == a).astype(jnp.float32)
        cnts = []
        for hh in range(BT // 256):
            hm = ((lane >= hh * 256) & (lane < (hh + 1) * 256)
                  ).astype(jnp.float32)
            c = jax.lax.dot_general(hm, oneh, (((1,), (1,)), ((), ())),
                                    preferred_element_type=jnp.float32)
            c16 = jnp.concatenate(
                [c, jnp.zeros((1, 16 - E), jnp.float32)], axis=1)
            cnts.append(c16.astype(jnp.int32))
        pieces.append(cnts)
    rows = []
    for hh in range(BT // 256):
        rows.append(jnp.concatenate(
            [pieces[0][hh].reshape(1, 1, 16), pieces[1][hh].reshape(1, 1, 16)],
            axis=1))
    hist_ref[...] = jnp.concatenate(rows, axis=0)


def _router_tc(x_flat, router_w):
    T, D = x_flat.shape
    E = router_w.shape[0]
    BT = min(512, T)
    NT = T // BT
    return pl.pallas_call(
        _router_body,
        grid=(NT,),
        in_specs=[
            pl.BlockSpec((BT, D), lambda t: (t, 0)),
            pl.BlockSpec((E, D), lambda t: (0, 0)),
        ],
        out_specs=[
            pl.BlockSpec((2, BT), lambda t: (0, t)),
            pl.BlockSpec((2, BT), lambda t: (0, t)),
            pl.BlockSpec((BT // 256, 2, 16), lambda t: (t, 0, 0)),
        ],
        out_shape=[
            jax.ShapeDtypeStruct((2, T), jnp.int32),
            jax.ShapeDtypeStruct((2, T), jnp.float32),
            jax.ShapeDtypeStruct((T // 256, 2, 16), jnp.int32),
        ],
    )(x_flat, router_w)


def _sc_mesh():
    info = plsc.get_sparse_core_info()
    nw = info.num_cores * info.num_subcores
    mesh = plsc.VectorSubcoreMesh(core_axis_name="c", subcore_axis_name="s")
    return info, nw, mesh


def _wid(info):
    return lax.axis_index("s") * info.num_cores + lax.axis_index("c")


def _hist_sc(eflat, E):
    A = eflat.shape[0]
    info, NW, mesh = _sc_mesh()
    CH = A // NW

    @functools.partial(
        pl.kernel, mesh=mesh,
        compiler_params=pltpu.CompilerParams(needs_layout_passes=False),
        out_type=jax.ShapeDtypeStruct((NW, 16), jnp.int32),
        scratch_types=[
            pltpu.VMEM((CH,), jnp.int32),
            pltpu.VMEM((16,), jnp.int32),
        ],
    )
    def hist_k(eidx_hbm, hist_hbm, eid_v, hbuf):
        w = _wid(info)
        pltpu.sync_copy(eidx_hbm.at[pl.ds(w * CH, CH)], eid_v)

        def step(i, accs):
            d = eid_v[pl.ds(i * 16, 16)]
            return tuple(a + (d == jnp.int32(e)).astype(jnp.int32)
                         for e, a in enumerate(accs))

        accs = lax.fori_loop(0, CH // 16, step,
                             tuple(jnp.zeros((16,), jnp.int32)
                                   for _ in range(E)))
        iota = lax.iota(jnp.int32, 16)
        hist = jnp.zeros((16,), jnp.int32)
        for e in range(E):
            hist = jnp.where(iota == e, jnp.sum(accs[e]), hist)
        hbuf[...] = hist
        pltpu.sync_copy(hbuf, hist_hbm.at[w])

    return hist_k(eflat)


def _plan_sc(eflat, gflat, hist, x_flat, T, E, P, G16):
    A = eflat.shape[0]
    D = x_flat.shape[1]
    info, NW, mesh = _sc_mesh()
    CH = A // NW
    NV = CH // 16
    RC = 32
    NCHK = CH // RC
    assert T % CH == 0

    @functools.partial(
        pl.kernel, mesh=mesh,
        compiler_params=pltpu.CompilerParams(needs_layout_passes=False),
        out_type=(
            jax.ShapeDtypeStruct((P, D), jnp.float32),
            jax.ShapeDtypeStruct((P,), jnp.float32),
            jax.ShapeDtypeStruct((G16,), jnp.int32),
            jax.ShapeDtypeStruct((A,), jnp.int32),
        ),
        scratch_types=[
            pltpu.VMEM((CH,), jnp.int32),
            pltpu.VMEM((CH,), jnp.float32),
            pltpu.VMEM((NW, 16), jnp.int32),
            pltpu.VMEM((NCHK, RC), jnp.int32),
            pltpu.VMEM((NCHK, RC), jnp.float32),
            pltpu.VMEM((CH,), jnp.int32),
            pltpu.VMEM((G16,), jnp.int32),
            pltpu.VMEM((16,), jnp.int32),
            pltpu.VMEM((2, RC, D), jnp.float32),
            pltpu.SemaphoreType.DMA,
            pltpu.SemaphoreType.DMA,
            pltpu.SemaphoreType.DMA,
        ],
    )
    def plan_k(eidx_hbm, gwt_hbm, hist_hbm, x_hbm, xs_hbm, gw_hbm, blk_hbm,
               pos_hbm, eid_v, gwv, hall, posb, gwb, pos_lin, blkb, runbuf,
               xrow, sem_s, sem_in, sem_out):
        w = _wid(info)
        base = w * CH
        pltpu.sync_copy(eidx_hbm.at[pl.ds(base, CH)], eid_v)
        pltpu.sync_copy(gwt_hbm.at[pl.ds(base, CH)], gwv)
        pltpu.sync_copy(hist_hbm, hall)
        iota = lax.iota(jnp.int32, 16)
        total = jnp.zeros((16,), jnp.int32)
        prior = jnp.zeros((16,), jnp.int32)
        for i in range(NW):
            h = hall[i, :]
            wi = (i % 2) * (NW // 2) + i // 2
            total = total + h
            prior = prior + jnp.where(jnp.int32(wi) < w, h, 0)
        padded = ((total + (BLK - 1)) // BLK) * BLK
        bases = plsc.cumsum(padded) - padded
        run = bases + prior

        @pl.when(w == 0)
        def _blocks():
            blkstart = bases // BLK
            blkcnt = padded // BLK
            for j in range(G16 // 16):
                bid = iota + j * 16
                acc = jnp.full((16,), -1, jnp.int32)
                for e in range(E):
                    s_e = jnp.sum(jnp.where(iota == e, blkstart, 0))
                    n_e = jnp.sum(jnp.where(iota == e, blkcnt, 0))
                    mb = (bid >= s_e) & (bid < s_e + n_e)
                    acc = jnp.where(mb, e, acc)
                blkb[pl.ds(j * 16, 16)] = acc
            pltpu.sync_copy(blkb, blk_hbm)

        for i in range(NV):
            d = eid_v[pl.ds(i * 16, 16)]
            runbuf[...] = run
            off = plsc.load_gather(runbuf, [d])
            pos_v = off
            for e in range(E):
                m = d == jnp.int32(e)
                mi = m.astype(jnp.int32)
                ranks = plsc.cumsum(mi) - mi
                pos_v = pos_v + jnp.where(m, ranks, 0)
                pc = plsc.all_reduce_population_count(m)
                run = run + jnp.where(iota == e, pc, 0)
            r, c = divmod(i * 16, RC)
            posb[r, pl.ds(c, 16)] = pos_v
            gwb[r, pl.ds(c, 16)] = gwv[pl.ds(i * 16, 16)]
            pos_lin[pl.ds(i * 16, 16)] = pos_v
        pltpu.sync_copy(pos_lin, pos_hbm.at[pl.ds(base, CH)])
        cps = [pltpu.async_copy(gwb.at[j], gw_hbm.at[posb.at[j]], sem_s)
               for j in range(NCHK)]
        tok0 = jnp.where(base < T, base, base - T)
        starts = []
        for c in range(min(2, NCHK)):
            starts.append(pltpu.async_copy(
                x_hbm.at[pl.ds(tok0 + c * RC, RC)], xrow.at[c % 2], sem_in))
        for c in range(NCHK):
            starts[c].wait()
            pltpu.async_copy(xrow.at[c % 2], xs_hbm.at[posb.at[c]],
                             sem_out).wait()
            if c + 2 < NCHK:
                starts.append(pltpu.async_copy(
                    x_hbm.at[pl.ds(tok0 + (c + 2) * RC, RC)],
                    xrow.at[c % 2], sem_in))
        for cp in cps:
            cp.wait()

    return plan_k(eflat, gflat, hist, x_flat)


def _gather_sc(src, idx):
    R, D = src.shape
    M = idx.shape[0]
    info, NW, mesh = _sc_mesh()
    RW = M // NW
    CH = min(32, RW)
    STEPS = RW // CH

    @functools.partial(
        pl.kernel, mesh=mesh,
        compiler_params=pltpu.CompilerParams(needs_layout_passes=False),
        out_type=jax.ShapeDtypeStruct((M, D), jnp.float32),
        scratch_types=[
            pltpu.VMEM((STEPS, CH), jnp.int32),
            pltpu.VMEM((2, CH, D), jnp.float32),
            pltpu.SemaphoreType.DMA,
            pltpu.SemaphoreType.DMA,
        ],
    )
    def gather_k(src_hbm, idx_hbm, out_hbm, idx_v, rows_v, sem_g, sem_o):
        w = _wid(info)
        for c in range(STEPS):
            pltpu.sync_copy(idx_hbm.at[pl.ds(w * RW + c * CH, CH)],
                            idx_v.at[c])
        gath = []
        for c in range(min(2, STEPS)):
            gath.append(pltpu.async_copy(src_hbm.at[idx_v.at[c]],
                                         rows_v.at[c % 2], sem_g))
        for c in range(STEPS):
            gath[c].wait()
            wb = pltpu.async_copy(rows_v.at[c % 2],
                                  out_hbm.at[pl.ds(w * RW + c * CH, CH)],
                                  sem_o)
            wb.wait()
            if c + 2 < STEPS:
                gath.append(pltpu.async_copy(src_hbm.at[idx_v.at[c + 2]],
                                             rows_v.at[c % 2], sem_g))

    return gather_k(src, idx)


def _mlp_body(bexp_ref, xs_ref, w1_ref, b1_ref, w2_ref, b2_ref, gw_ref,
              os_ref):
    b = pl.program_id(0)
    e = bexp_ref[b]

    @pl.when(e >= 0)
    def _():
        gw = gw_ref[...]
        h = jax.lax.dot_general(
            xs_ref[...], w1_ref[0],
            (((1,), (1,)), ((), ())), preferred_element_type=jnp.float32)
        h = jnp.maximum(h + b1_ref[0], 0.0) * gw
        os_ref[...] = jax.lax.dot_general(
            h, w2_ref[0], (((1,), (1,)), ((), ())),
            preferred_element_type=jnp.float32) + gw * b2_ref[0]


def _mlp_tc(xs, blk_exp, gw_sorted, w1, b1, w2, b2):
    P, D = xs.shape
    E, F, _ = w1.shape
    G = P // BLK
    grid_spec = pltpu.PrefetchScalarGridSpec(
        num_scalar_prefetch=1,
        grid=(G,),
        in_specs=[
            pl.BlockSpec((BLK, D), lambda b, s: (b, 0)),
            pl.BlockSpec((1, F, D), lambda b, s: (jnp.maximum(s[b], 0), 0, 0)),
            pl.BlockSpec((1, 1, F), lambda b, s: (jnp.maximum(s[b], 0), 0, 0)),
            pl.BlockSpec((1, D, F), lambda b, s: (jnp.maximum(s[b], 0), 0, 0)),
            pl.BlockSpec((1, 1, D), lambda b, s: (jnp.maximum(s[b], 0), 0, 0)),
            pl.BlockSpec((BLK, 1), lambda b, s: (b, 0)),
        ],
        out_specs=pl.BlockSpec((BLK, D), lambda b, s: (b, 0)),
    )
    return pl.pallas_call(
        _mlp_body,
        grid_spec=grid_spec,
        out_shape=jax.ShapeDtypeStruct((P, D), jnp.float32),
    )(blk_exp, xs, w1, b1.reshape(E, 1, F),
      w2, b2.reshape(E, 1, D),
      gw_sorted.reshape(P, 1))


def _combine_sc(os_, pos, T):
    P, D = os_.shape
    info, NW, mesh = _sc_mesh()
    TW = T // NW
    RC = 16
    STEPS = TW // RC

    @functools.partial(
        pl.kernel, mesh=mesh,
        compiler_params=pltpu.CompilerParams(needs_layout_passes=False),
        out_type=jax.ShapeDtypeStruct((T, D), jnp.float32),
        scratch_types=[
            pltpu.VMEM((STEPS, RC), jnp.int32),
            pltpu.VMEM((STEPS, RC), jnp.int32),
            pltpu.VMEM((2, RC, D), jnp.float32),
            pltpu.VMEM((2, RC, D), jnp.float32),
            pltpu.SemaphoreType.DMA,
            pltpu.SemaphoreType.DMA,
            pltpu.SemaphoreType.DMA,
        ],
    )
    def comb_k(os_hbm, pos_hbm, out_hbm, idxa, idxb, bufa, bufb,
               sem_a, sem_b, sem_o):
        w = _wid(info)
        tok0 = w * TW
        for c in range(STEPS):
            pltpu.sync_copy(pos_hbm.at[pl.ds(tok0 + c * RC, RC)],
                            idxa.at[c])
            pltpu.sync_copy(pos_hbm.at[pl.ds(T + tok0 + c * RC, RC)],
                            idxb.at[c])
        ga, gb = [], []
        for c in range(min(2, STEPS)):
            ga.append(pltpu.async_copy(os_hbm.at[idxa.at[c]],
                                       bufa.at[c % 2], sem_a))
            gb.append(pltpu.async_copy(os_hbm.at[idxb.at[c]],
                                       bufb.at[c % 2], sem_b))
        for c in range(STEPS):
            cc = c % 2
            ga[c].wait()
            gb[c].wait()
            for r in range(RC):
                def vstep(v, _, cc=cc, r=r):
                    sl = pl.ds(v * 16, 16)
                    bufa[cc, r, sl] = bufa[cc, r, sl] + bufb[cc, r, sl]
                    return 0
                lax.fori_loop(0, D // 16, vstep, 0)
            pltpu.async_copy(bufa.at[cc],
                             out_hbm.at[pl.ds(tok0 + c * RC, RC)],
                             sem_o).wait()
            if c + 2 < STEPS:
                ga.append(pltpu.async_copy(os_hbm.at[idxa.at[c + 2]],
                                           bufa.at[cc], sem_a))
                gb.append(pltpu.async_copy(os_hbm.at[idxb.at[c + 2]],
                                           bufb.at[cc], sem_b))

    return comb_k(os_, pos)


def _add_body(a_ref, b_ref, o_ref):
    o_ref[...] = a_ref[0] + b_ref[0]


def _add_tc(comb):
    _, T, D = comb.shape
    BT = min(1024, T)
    NT = T // BT
    return pl.pallas_call(
        _add_body,
        grid=(NT,),
        in_specs=[
            pl.BlockSpec((1, BT, D), lambda t: (0, t, 0)),
            pl.BlockSpec((1, BT, D), lambda t: (1, t, 0)),
        ],
        out_specs=pl.BlockSpec((BT, D), lambda t: (t, 0)),
        out_shape=jax.ShapeDtypeStruct((T, D), jnp.float32),
    )(comb, comb)


def kernel(x, router_w, w1, b1, w2, b2):
    B, N, D = x.shape
    E = router_w.shape[0]
    T = B * N
    A = 2 * T
    P = A + E * BLK
    G16 = ((P // BLK + 15) // 16) * 16
    x_flat = x.reshape(T, D)
    eidx, gwt, hist3 = _router_tc(x_flat, router_w)
    eflat = eidx.reshape(A)
    gflat = gwt.reshape(A)
    hist = hist3.reshape(T // 256 * 2, 16)
    xs, gw_s, blk_e, pos = _plan_sc(eflat, gflat, hist, x_flat, T, E, P, G16)
    os_ = _mlp_tc(xs, blk_e, gw_s, w1, b1, w2, b2)
    comb = _gather_sc(os_, pos).reshape(2, T, D)
    out = _add_tc(comb)
    return out.reshape(B, N, D)
